# baseline (speedup 1.0000x reference)
import functools

import jax
import jax.numpy as jnp
from jax import lax
from jax.experimental import pallas as pl
from jax.experimental.pallas import tpu as pltpu
from jax.experimental.pallas import tpu_sc as plsc

N_NODES = 10000
N_EDGES = 320000
D_IN = 128
D_HID = 64
D_OUT = 128

NC = 2
NS = 16
NW = NC * NS
EPW = N_EDGES // NW
CHUNK = 125
NCHUNK = EPW // CHUNK
NBUF = 2
N_PAD = 10240
ROWS_PER_SUB = N_PAD // NS

_mesh = plsc.VectorSubcoreMesh(core_axis_name="c", subcore_axis_name="s")


def _seg_body(y_hbm, src_hbm, dst_hbm, z64_hbm, z16_hbm, ones_hbm,
              out_hbm, cnt_hbm, src_v, dst_v, rows_v, ones_v,
              acc_sh, cnt_sh, sems, with_counts):
    cid = lax.axis_index("c")
    sid = lax.axis_index("s")
    wid = sid * NC + cid

    pltpu.sync_copy(z64_hbm, acc_sh.at[pl.ds(sid * ROWS_PER_SUB, ROWS_PER_SUB)])
    if with_counts:
        pltpu.sync_copy(z16_hbm, cnt_sh.at[pl.ds(sid * ROWS_PER_SUB, ROWS_PER_SUB)])
        pltpu.sync_copy(ones_hbm, ones_v)
    pltpu.sync_copy(src_hbm.at[wid], src_v)
    pltpu.sync_copy(dst_hbm.at[wid], dst_v)
    plsc.subcore_barrier()

    for b in range(NBUF):
        pltpu.async_copy(y_hbm.at[src_v.at[b]], rows_v.at[b], sems[b])

    def body(i, carry):
        c = i * NBUF
        for b in range(NBUF):
            ch = c + b
            pltpu.make_async_copy(y_hbm.at[src_v.at[ch]], rows_v.at[b],
                                  sems[b]).wait()
            pltpu.sync_copy(rows_v.at[b], acc_sh.at[dst_v.at[ch]], add=True)
            if with_counts:
                pltpu.sync_copy(ones_v, cnt_sh.at[dst_v.at[ch]], add=True)
            pltpu.async_copy(y_hbm.at[src_v.at[ch + NBUF]], rows_v.at[b],
                             sems[b])
        return carry

    lax.fori_loop(0, NCHUNK // NBUF, body, 0)
    for b in range(NBUF):
        pltpu.make_async_copy(y_hbm.at[src_v.at[NCHUNK + b]], rows_v.at[b],
                              sems[b]).wait()
    plsc.subcore_barrier()

    row0 = sid * ROWS_PER_SUB
    pltpu.sync_copy(acc_sh.at[pl.ds(row0, ROWS_PER_SUB)],
                    out_hbm.at[cid, pl.ds(row0, ROWS_PER_SUB)])
    if with_counts:
        pltpu.sync_copy(cnt_sh.at[pl.ds(row0, ROWS_PER_SUB)],
                        cnt_hbm.at[cid, pl.ds(row0, ROWS_PER_SUB)])


@functools.partial(
    pl.kernel,
    out_type=(jax.ShapeDtypeStruct((NC, N_PAD, D_HID), jnp.float32),
              jax.ShapeDtypeStruct((NC, N_PAD, 16), jnp.float32)),
    mesh=_mesh,
    scratch_types=[
        pltpu.VMEM((NCHUNK + NBUF, CHUNK), jnp.int32),
        pltpu.VMEM((NCHUNK, CHUNK), jnp.int32),
        pltpu.VMEM((NBUF, CHUNK, D_HID), jnp.float32),
        pltpu.VMEM((CHUNK, 16), jnp.float32),
        pltpu.VMEM_SHARED((N_PAD, D_HID), jnp.float32),
        pltpu.VMEM_SHARED((N_PAD, 16), jnp.float32),
        pltpu.SemaphoreType.DMA,
        pltpu.SemaphoreType.DMA,
    ],
    compiler_params=pltpu.CompilerParams(use_tc_tiling_on_sc=False),
)
def _sc_seg_counts(y_hbm, src_hbm, dst_hbm, z64_hbm, z16_hbm, ones_hbm,
                   out_hbm, cnt_hbm, src_v, dst_v, rows_v, ones_v,
                   acc_sh, cnt_sh, sem0, sem1):
    _seg_body(y_hbm, src_hbm, dst_hbm, z64_hbm, z16_hbm, ones_hbm,
              out_hbm, cnt_hbm, src_v, dst_v, rows_v, ones_v,
              acc_sh, cnt_sh, [sem0, sem1], with_counts=True)


@functools.partial(
    pl.kernel,
    out_type=jax.ShapeDtypeStruct((NC, N_PAD, D_HID), jnp.float32),
    mesh=_mesh,
    scratch_types=[
        pltpu.VMEM((NCHUNK + NBUF, CHUNK), jnp.int32),
        pltpu.VMEM((NCHUNK, CHUNK), jnp.int32),
        pltpu.VMEM((NBUF, CHUNK, D_HID), jnp.float32),
        pltpu.VMEM_SHARED((N_PAD, D_HID), jnp.float32),
        pltpu.SemaphoreType.DMA,
        pltpu.SemaphoreType.DMA,
    ],
    compiler_params=pltpu.CompilerParams(use_tc_tiling_on_sc=False),
)
def _sc_seg(y_hbm, src_hbm, dst_hbm, z64_hbm, out_hbm,
            src_v, dst_v, rows_v, acc_sh, sem0, sem1):
    _seg_body(y_hbm, src_hbm, dst_hbm, z64_hbm, None, None,
              out_hbm, None, src_v, dst_v, rows_v, None,
              acc_sh, None, [sem0, sem1], with_counts=False)


def _tc_a_body(x_ref, wl1_ref, wr1_ref, bl1_ref, y1_ref, r1_ref):
    x = x_ref[...]
    dn = (((1,), (1,)), ((), ()))
    y1_ref[...] = lax.dot_general(x, wl1_ref[...], dn,
                                  preferred_element_type=jnp.float32)
    r1_ref[...] = lax.dot_general(x, wr1_ref[...], dn,
                                  preferred_element_type=jnp.float32) + bl1_ref[...]


_tc_a = pl.pallas_call(
    _tc_a_body,
    out_shape=(jax.ShapeDtypeStruct((N_NODES, D_HID), jnp.float32),
               jax.ShapeDtypeStruct((N_NODES, D_HID), jnp.float32)),
)


def _tc_b_body(s1_ref, cnt_ref, r1_ref, wr2_ref, bl2_ref, h_ref, r2_ref):
    s = (s1_ref[0] + s1_ref[1])[:N_NODES]
    c = (cnt_ref[0, :, 0:1] + cnt_ref[1, :, 0:1])[:N_NODES]
    agg = s / jnp.maximum(c, 1.0)
    h = jnp.maximum(agg + r1_ref[...], 0.0)
    h_ref[...] = h
    dn = (((1,), (1,)), ((), ()))
    r2_ref[...] = lax.dot_general(h, wr2_ref[...], dn,
                                  preferred_element_type=jnp.float32) + bl2_ref[...]


_tc_b = pl.pallas_call(
    _tc_b_body,
    out_shape=(jax.ShapeDtypeStruct((N_NODES, D_HID), jnp.float32),
               jax.ShapeDtypeStruct((N_NODES, D_OUT), jnp.float32)),
)


def _tc_c_body(s2_ref, cnt_ref, r2_ref, wl2_ref, out_ref):
    s = (s2_ref[0] + s2_ref[1])[:N_NODES]
    c = (cnt_ref[0, :, 0:1] + cnt_ref[1, :, 0:1])[:N_NODES]
    agg = s / jnp.maximum(c, 1.0)
    dn = (((1,), (1,)), ((), ()))
    z = lax.dot_general(agg, wl2_ref[...], dn,
                        preferred_element_type=jnp.float32) + r2_ref[...]
    m = jnp.max(z, axis=1, keepdims=True)
    lse = jnp.log(jnp.sum(jnp.exp(z - m), axis=1, keepdims=True)) + m
    out_ref[...] = z - lse


_tc_c = pl.pallas_call(
    _tc_c_body,
    out_shape=jax.ShapeDtypeStruct((N_NODES, D_OUT), jnp.float32),
)


def kernel(x, edge_index, Wl1, bl1, Wr1, Wl2, bl2, Wr2):
    ei = edge_index.astype(jnp.int32)
    src3 = jnp.concatenate(
        [ei[0].reshape(NW, NCHUNK, CHUNK),
         jnp.zeros((NW, NBUF, CHUNK), jnp.int32)], axis=1)
    dst3 = ei[1].reshape(NW, NCHUNK, CHUNK)
    z64 = jnp.zeros((ROWS_PER_SUB, D_HID), jnp.float32)
    z16 = jnp.zeros((ROWS_PER_SUB, 16), jnp.float32)
    ones16 = jnp.ones((CHUNK, 16), jnp.float32)

    y1, r1 = _tc_a(x, Wl1, Wr1, bl1.reshape(1, D_HID))
    s1, cntw = _sc_seg_counts(y1, src3, dst3, z64, z16, ones16)
    h, r2 = _tc_b(s1, cntw, r1, Wr2, bl2.reshape(1, D_OUT))
    s2 = _sc_seg(h, src3, dst3, z64)
    return _tc_c(s2, cntw, r2, Wl2)


# speedup vs baseline: 1.9872x; 1.9872x over previous
import functools

import jax
import jax.numpy as jnp
from jax import lax
from jax.experimental import pallas as pl
from jax.experimental.pallas import tpu as pltpu
from jax.experimental.pallas import tpu_sc as plsc

N_NODES = 10000
N_EDGES = 320000
D_IN = 128
D_HID = 64
D_OUT = 128

NC = 2
NS = 16
NW = NC * NS
EPW = N_EDGES // NW
GROUP = 500
NCHUNK = EPW // GROUP
N_PAD = 10240
ROWS_PER_SUB = N_PAD // NS

_mesh = plsc.VectorSubcoreMesh(core_axis_name="c", subcore_axis_name="s")


def _seg_body(y_hbm, src_hbm, dst_hbm, z64_hbm, z16_hbm, ones_hbm,
              out_hbm, cnt_hbm, src_v, dst_v, rows_v, ones_v,
              acc_sh, cnt_sh, sems, with_counts):
    cid = lax.axis_index("c")
    sid = lax.axis_index("s")
    wid = sid * NC + cid

    pltpu.sync_copy(z64_hbm, acc_sh.at[pl.ds(sid * ROWS_PER_SUB, ROWS_PER_SUB)])
    if with_counts:
        pltpu.sync_copy(z16_hbm, cnt_sh.at[pl.ds(sid * ROWS_PER_SUB, ROWS_PER_SUB)])
        pltpu.sync_copy(ones_hbm, ones_v)
    pltpu.sync_copy(src_hbm.at[wid], src_v)
    pltpu.sync_copy(dst_hbm.at[wid], dst_v)
    plsc.subcore_barrier()

    def body(c, carry):
        pltpu.async_copy(y_hbm.at[src_v.at[c]], rows_v, sems[0]).wait()
        pltpu.sync_copy(rows_v, acc_sh.at[dst_v.at[c]], add=True)
        if with_counts:
            pltpu.sync_copy(ones_v, cnt_sh.at[dst_v.at[c]], add=True)
        return carry

    lax.fori_loop(0, NCHUNK, body, 0)
    plsc.subcore_barrier()

    row0 = sid * ROWS_PER_SUB
    pltpu.sync_copy(acc_sh.at[pl.ds(row0, ROWS_PER_SUB)],
                    out_hbm.at[cid, pl.ds(row0, ROWS_PER_SUB)])
    if with_counts:
        pltpu.sync_copy(cnt_sh.at[pl.ds(row0, ROWS_PER_SUB)],
                        cnt_hbm.at[cid, pl.ds(row0, ROWS_PER_SUB)])


@functools.partial(
    pl.kernel,
    out_type=(jax.ShapeDtypeStruct((NC, N_PAD, D_HID), jnp.float32),
              jax.ShapeDtypeStruct((NC, N_PAD, 16), jnp.float32)),
    mesh=_mesh,
    scratch_types=[
        pltpu.VMEM((NCHUNK, GROUP), jnp.int32),
        pltpu.VMEM((NCHUNK, GROUP), jnp.int32),
        pltpu.VMEM((GROUP, D_HID), jnp.float32),
        pltpu.VMEM((GROUP, 16), jnp.float32),
        pltpu.VMEM_SHARED((N_PAD, D_HID), jnp.float32),
        pltpu.VMEM_SHARED((N_PAD, 16), jnp.float32),
        pltpu.SemaphoreType.DMA,
        pltpu.SemaphoreType.DMA,
    ],
    compiler_params=pltpu.CompilerParams(use_tc_tiling_on_sc=False),
)
def _sc_seg_counts(y_hbm, src_hbm, dst_hbm, z64_hbm, z16_hbm, ones_hbm,
                   out_hbm, cnt_hbm, src_v, dst_v, rows_v, ones_v,
                   acc_sh, cnt_sh, sem0, sem1):
    _seg_body(y_hbm, src_hbm, dst_hbm, z64_hbm, z16_hbm, ones_hbm,
              out_hbm, cnt_hbm, src_v, dst_v, rows_v, ones_v,
              acc_sh, cnt_sh, [sem0, sem1], with_counts=True)


@functools.partial(
    pl.kernel,
    out_type=jax.ShapeDtypeStruct((NC, N_PAD, D_HID), jnp.float32),
    mesh=_mesh,
    scratch_types=[
        pltpu.VMEM((NCHUNK, GROUP), jnp.int32),
        pltpu.VMEM((NCHUNK, GROUP), jnp.int32),
        pltpu.VMEM((GROUP, D_HID), jnp.float32),
        pltpu.VMEM_SHARED((N_PAD, D_HID), jnp.float32),
        pltpu.SemaphoreType.DMA,
        pltpu.SemaphoreType.DMA,
    ],
    compiler_params=pltpu.CompilerParams(use_tc_tiling_on_sc=False),
)
def _sc_seg(y_hbm, src_hbm, dst_hbm, z64_hbm, out_hbm,
            src_v, dst_v, rows_v, acc_sh, sem0, sem1):
    _seg_body(y_hbm, src_hbm, dst_hbm, z64_hbm, None, None,
              out_hbm, None, src_v, dst_v, rows_v, None,
              acc_sh, None, [sem0, sem1], with_counts=False)


def _tc_a_body(x_ref, wl1_ref, wr1_ref, bl1_ref, y1_ref, r1_ref):
    x = x_ref[...]
    dn = (((1,), (1,)), ((), ()))
    y1_ref[...] = lax.dot_general(x, wl1_ref[...], dn,
                                  preferred_element_type=jnp.float32)
    r1_ref[...] = lax.dot_general(x, wr1_ref[...], dn,
                                  preferred_element_type=jnp.float32) + bl1_ref[...]


_tc_a = pl.pallas_call(
    _tc_a_body,
    out_shape=(jax.ShapeDtypeStruct((N_NODES, D_HID), jnp.float32),
               jax.ShapeDtypeStruct((N_NODES, D_HID), jnp.float32)),
)


def _tc_b_body(s1_ref, cnt_ref, r1_ref, wr2_ref, bl2_ref, h_ref, r2_ref):
    s = (s1_ref[0] + s1_ref[1])[:N_NODES]
    c = (cnt_ref[0, :, 0:1] + cnt_ref[1, :, 0:1])[:N_NODES]
    agg = s / jnp.maximum(c, 1.0)
    h = jnp.maximum(agg + r1_ref[...], 0.0)
    h_ref[...] = h
    dn = (((1,), (1,)), ((), ()))
    r2_ref[...] = lax.dot_general(h, wr2_ref[...], dn,
                                  preferred_element_type=jnp.float32) + bl2_ref[...]


_tc_b = pl.pallas_call(
    _tc_b_body,
    out_shape=(jax.ShapeDtypeStruct((N_NODES, D_HID), jnp.float32),
               jax.ShapeDtypeStruct((N_NODES, D_OUT), jnp.float32)),
)


def _tc_c_body(s2_ref, cnt_ref, r2_ref, wl2_ref, out_ref):
    s = (s2_ref[0] + s2_ref[1])[:N_NODES]
    c = (cnt_ref[0, :, 0:1] + cnt_ref[1, :, 0:1])[:N_NODES]
    agg = s / jnp.maximum(c, 1.0)
    dn = (((1,), (1,)), ((), ()))
    z = lax.dot_general(agg, wl2_ref[...], dn,
                        preferred_element_type=jnp.float32) + r2_ref[...]
    m = jnp.max(z, axis=1, keepdims=True)
    lse = jnp.log(jnp.sum(jnp.exp(z - m), axis=1, keepdims=True)) + m
    out_ref[...] = z - lse


_tc_c = pl.pallas_call(
    _tc_c_body,
    out_shape=jax.ShapeDtypeStruct((N_NODES, D_OUT), jnp.float32),
)


def kernel(x, edge_index, Wl1, bl1, Wr1, Wl2, bl2, Wr2):
    ei = edge_index.astype(jnp.int32)
    src3 = ei[0].reshape(NW, NCHUNK, GROUP)
    dst3 = ei[1].reshape(NW, NCHUNK, GROUP)
    z64 = jnp.zeros((ROWS_PER_SUB, D_HID), jnp.float32)
    z16 = jnp.zeros((ROWS_PER_SUB, 16), jnp.float32)
    ones16 = jnp.ones((GROUP, 16), jnp.float32)

    y1, r1 = _tc_a(x, Wl1, Wr1, bl1.reshape(1, D_HID))
    s1, cntw = _sc_seg_counts(y1, src3, dst3, z64, z16, ones16)
    h, r2 = _tc_b(s1, cntw, r1, Wr2, bl2.reshape(1, D_OUT))
    s2 = _sc_seg(h, src3, dst3, z64)
    return _tc_c(s2, cntw, r2, Wl2)


# speedup vs baseline: 2.1396x; 1.0767x over previous
import functools

import jax
import jax.numpy as jnp
from jax import lax
from jax.experimental import pallas as pl
from jax.experimental.pallas import tpu as pltpu
from jax.experimental.pallas import tpu_sc as plsc

N_NODES = 10000
N_EDGES = 320000
D_IN = 128
D_HID = 64
D_OUT = 128

NC = 2
NS = 16
NW = NC * NS
EPW = N_EDGES // NW
GROUP = 250
NCHUNK = EPW // GROUP
N_PAD = 10240
ROWS_PER_SUB = N_PAD // NS

_mesh = plsc.VectorSubcoreMesh(core_axis_name="c", subcore_axis_name="s")


def _seg_body(y_hbm, src_hbm, dst_hbm, z64_hbm, z16_hbm, ones_hbm,
              out_hbm, cnt_hbm, src_v, dst_v, rows_v, ones_v,
              acc_sh, cnt_sh, sems, with_counts):
    cid = lax.axis_index("c")
    sid = lax.axis_index("s")
    wid = sid * NC + cid

    pltpu.sync_copy(z64_hbm, acc_sh.at[pl.ds(sid * ROWS_PER_SUB, ROWS_PER_SUB)])
    if with_counts:
        pltpu.sync_copy(z16_hbm, cnt_sh.at[pl.ds(sid * ROWS_PER_SUB, ROWS_PER_SUB)])
        pltpu.sync_copy(ones_hbm, ones_v)
    pltpu.sync_copy(src_hbm.at[wid], src_v)
    pltpu.sync_copy(dst_hbm.at[wid], dst_v)
    plsc.subcore_barrier()

    def body(i, carry):
        for b in range(2):
            c = i * 2 + b
            cp = jnp.maximum(c - 2, 0)

            @pl.when(c >= 2)
            def _():
                pltpu.make_async_copy(rows_v.at[b], acc_sh.at[dst_v.at[cp]],
                                      sems[b]).wait()

            pltpu.async_copy(y_hbm.at[src_v.at[c]], rows_v.at[b],
                             sems[2]).wait()
            pltpu.async_copy(rows_v.at[b], acc_sh.at[dst_v.at[c]], sems[b],
                             add=True)
            if with_counts:
                pltpu.sync_copy(ones_v, cnt_sh.at[dst_v.at[c]], add=True)
        return carry

    lax.fori_loop(0, NCHUNK // 2, body, 0)
    for b in range(2):
        pltpu.make_async_copy(rows_v.at[b],
                              acc_sh.at[dst_v.at[NCHUNK - 2 + b]],
                              sems[b]).wait()
    plsc.subcore_barrier()

    row0 = sid * ROWS_PER_SUB
    pltpu.sync_copy(acc_sh.at[pl.ds(row0, ROWS_PER_SUB)],
                    out_hbm.at[cid, pl.ds(row0, ROWS_PER_SUB)])
    if with_counts:
        pltpu.sync_copy(cnt_sh.at[pl.ds(row0, ROWS_PER_SUB)],
                        cnt_hbm.at[cid, pl.ds(row0, ROWS_PER_SUB)])


@functools.partial(
    pl.kernel,
    out_type=(jax.ShapeDtypeStruct((NC, N_PAD, D_HID), jnp.float32),
              jax.ShapeDtypeStruct((NC, N_PAD, 16), jnp.float32)),
    mesh=_mesh,
    scratch_types=[
        pltpu.VMEM((NCHUNK, GROUP), jnp.int32),
        pltpu.VMEM((NCHUNK, GROUP), jnp.int32),
        pltpu.VMEM((2, GROUP, D_HID), jnp.float32),
        pltpu.VMEM((GROUP, 16), jnp.float32),
        pltpu.VMEM_SHARED((N_PAD, D_HID), jnp.float32),
        pltpu.VMEM_SHARED((N_PAD, 16), jnp.float32),
        pltpu.SemaphoreType.DMA,
        pltpu.SemaphoreType.DMA,
        pltpu.SemaphoreType.DMA,
    ],
    compiler_params=pltpu.CompilerParams(use_tc_tiling_on_sc=False),
)
def _sc_seg_counts(y_hbm, src_hbm, dst_hbm, z64_hbm, z16_hbm, ones_hbm,
                   out_hbm, cnt_hbm, src_v, dst_v, rows_v, ones_v,
                   acc_sh, cnt_sh, sem0, sem1, sem2):
    _seg_body(y_hbm, src_hbm, dst_hbm, z64_hbm, z16_hbm, ones_hbm,
              out_hbm, cnt_hbm, src_v, dst_v, rows_v, ones_v,
              acc_sh, cnt_sh, [sem0, sem1, sem2], with_counts=True)


@functools.partial(
    pl.kernel,
    out_type=jax.ShapeDtypeStruct((NC, N_PAD, D_HID), jnp.float32),
    mesh=_mesh,
    scratch_types=[
        pltpu.VMEM((NCHUNK, GROUP), jnp.int32),
        pltpu.VMEM((NCHUNK, GROUP), jnp.int32),
        pltpu.VMEM((2, GROUP, D_HID), jnp.float32),
        pltpu.VMEM_SHARED((N_PAD, D_HID), jnp.float32),
        pltpu.SemaphoreType.DMA,
        pltpu.SemaphoreType.DMA,
        pltpu.SemaphoreType.DMA,
    ],
    compiler_params=pltpu.CompilerParams(use_tc_tiling_on_sc=False),
)
def _sc_seg(y_hbm, src_hbm, dst_hbm, z64_hbm, out_hbm,
            src_v, dst_v, rows_v, acc_sh, sem0, sem1, sem2):
    _seg_body(y_hbm, src_hbm, dst_hbm, z64_hbm, None, None,
              out_hbm, None, src_v, dst_v, rows_v, None,
              acc_sh, None, [sem0, sem1, sem2], with_counts=False)


def _tc_a_body(x_ref, wl1_ref, wr1_ref, bl1_ref, y1_ref, r1_ref):
    x = x_ref[...]
    dn = (((1,), (1,)), ((), ()))
    y1_ref[...] = lax.dot_general(x, wl1_ref[...], dn,
                                  preferred_element_type=jnp.float32)
    r1_ref[...] = lax.dot_general(x, wr1_ref[...], dn,
                                  preferred_element_type=jnp.float32) + bl1_ref[...]


_tc_a = pl.pallas_call(
    _tc_a_body,
    out_shape=(jax.ShapeDtypeStruct((N_NODES, D_HID), jnp.float32),
               jax.ShapeDtypeStruct((N_NODES, D_HID), jnp.float32)),
)


def _tc_b_body(s1_ref, cnt_ref, r1_ref, wr2_ref, bl2_ref, h_ref, r2_ref):
    s = (s1_ref[0] + s1_ref[1])[:N_NODES]
    c = (cnt_ref[0, :, 0:1] + cnt_ref[1, :, 0:1])[:N_NODES]
    agg = s / jnp.maximum(c, 1.0)
    h = jnp.maximum(agg + r1_ref[...], 0.0)
    h_ref[...] = h
    dn = (((1,), (1,)), ((), ()))
    r2_ref[...] = lax.dot_general(h, wr2_ref[...], dn,
                                  preferred_element_type=jnp.float32) + bl2_ref[...]


_tc_b = pl.pallas_call(
    _tc_b_body,
    out_shape=(jax.ShapeDtypeStruct((N_NODES, D_HID), jnp.float32),
               jax.ShapeDtypeStruct((N_NODES, D_OUT), jnp.float32)),
)


def _tc_c_body(s2_ref, cnt_ref, r2_ref, wl2_ref, out_ref):
    s = (s2_ref[0] + s2_ref[1])[:N_NODES]
    c = (cnt_ref[0, :, 0:1] + cnt_ref[1, :, 0:1])[:N_NODES]
    agg = s / jnp.maximum(c, 1.0)
    dn = (((1,), (1,)), ((), ()))
    z = lax.dot_general(agg, wl2_ref[...], dn,
                        preferred_element_type=jnp.float32) + r2_ref[...]
    m = jnp.max(z, axis=1, keepdims=True)
    lse = jnp.log(jnp.sum(jnp.exp(z - m), axis=1, keepdims=True)) + m
    out_ref[...] = z - lse


_tc_c = pl.pallas_call(
    _tc_c_body,
    out_shape=jax.ShapeDtypeStruct((N_NODES, D_OUT), jnp.float32),
)


def kernel(x, edge_index, Wl1, bl1, Wr1, Wl2, bl2, Wr2):
    ei = edge_index.astype(jnp.int32)
    src3 = ei[0].reshape(NW, NCHUNK, GROUP)
    dst3 = ei[1].reshape(NW, NCHUNK, GROUP)
    z64 = jnp.zeros((ROWS_PER_SUB, D_HID), jnp.float32)
    z16 = jnp.zeros((ROWS_PER_SUB, 16), jnp.float32)
    ones16 = jnp.ones((GROUP, 16), jnp.float32)

    y1, r1 = _tc_a(x, Wl1, Wr1, bl1.reshape(1, D_HID))
    s1, cntw = _sc_seg_counts(y1, src3, dst3, z64, z16, ones16)
    h, r2 = _tc_b(s1, cntw, r1, Wr2, bl2.reshape(1, D_OUT))
    s2 = _sc_seg(h, src3, dst3, z64)
    return _tc_c(s2, cntw, r2, Wl2)


# speedup vs baseline: 2.2050x; 1.0306x over previous
import functools

import jax
import jax.numpy as jnp
from jax import lax
from jax.experimental import pallas as pl
from jax.experimental.pallas import tpu as pltpu
from jax.experimental.pallas import tpu_sc as plsc

N_NODES = 10000
N_EDGES = 320000
D_IN = 128
D_HID = 64
D_OUT = 128

NC = 2
NS = 16
NW = NC * NS
EPW = N_EDGES // NW
GROUP1 = 250
NCHUNK1 = EPW // GROUP1
GROUP2 = 500
NCHUNK2 = EPW // GROUP2
N_PAD = 10240
ROWS_PER_SUB = N_PAD // NS

_mesh = plsc.VectorSubcoreMesh(core_axis_name="c", subcore_axis_name="s")


def _seg_body(y_hbm, src_hbm, dst_hbm, z64_hbm, z16_hbm, ones_hbm,
              out_hbm, cnt_hbm, src_v, dst_v, rows_v, ones_v,
              acc_sh, cnt_sh, sems, nchunk, with_counts):
    cid = lax.axis_index("c")
    sid = lax.axis_index("s")
    wid = sid * NC + cid
    row0 = sid * ROWS_PER_SUB

    stage = [(z64_hbm, acc_sh.at[pl.ds(row0, ROWS_PER_SUB)]),
             (src_hbm.at[wid], src_v),
             (dst_hbm.at[wid], dst_v)]
    if with_counts:
        stage += [(z16_hbm, cnt_sh.at[pl.ds(row0, ROWS_PER_SUB)]),
                  (ones_hbm, ones_v)]
    for s, d in stage:
        pltpu.async_copy(s, d, sems[2])
    for s, d in stage:
        pltpu.make_async_copy(s, d, sems[2]).wait()
    plsc.subcore_barrier()

    def body(i, carry):
        for b in range(2):
            c = i * 2 + b
            cp = jnp.maximum(c - 2, 0)

            @pl.when(c >= 2)
            def _():
                pltpu.make_async_copy(rows_v.at[b], acc_sh.at[dst_v.at[cp]],
                                      sems[b]).wait()

            pltpu.async_copy(y_hbm.at[src_v.at[c]], rows_v.at[b],
                             sems[2]).wait()
            pltpu.async_copy(rows_v.at[b], acc_sh.at[dst_v.at[c]], sems[b],
                             add=True)
            if with_counts:
                pltpu.sync_copy(ones_v, cnt_sh.at[dst_v.at[c]], add=True)
        return carry

    lax.fori_loop(0, nchunk // 2, body, 0)
    for b in range(2):
        pltpu.make_async_copy(rows_v.at[b],
                              acc_sh.at[dst_v.at[nchunk - 2 + b]],
                              sems[b]).wait()
    plsc.subcore_barrier()

    out = [(acc_sh.at[pl.ds(row0, ROWS_PER_SUB)],
            out_hbm.at[cid, pl.ds(row0, ROWS_PER_SUB)])]
    if with_counts:
        out += [(cnt_sh.at[pl.ds(row0, ROWS_PER_SUB)],
                 cnt_hbm.at[cid, pl.ds(row0, ROWS_PER_SUB)])]
    for s, d in out:
        pltpu.async_copy(s, d, sems[2])
    for s, d in out:
        pltpu.make_async_copy(s, d, sems[2]).wait()


@functools.partial(
    pl.kernel,
    out_type=(jax.ShapeDtypeStruct((NC, N_PAD, D_HID), jnp.float32),
              jax.ShapeDtypeStruct((NC, N_PAD, 16), jnp.float32)),
    mesh=_mesh,
    scratch_types=[
        pltpu.VMEM((NCHUNK1, GROUP1), jnp.int32),
        pltpu.VMEM((NCHUNK1, GROUP1), jnp.int32),
        pltpu.VMEM((2, GROUP1, D_HID), jnp.float32),
        pltpu.VMEM((GROUP1, 16), jnp.float32),
        pltpu.VMEM_SHARED((N_PAD, D_HID), jnp.float32),
        pltpu.VMEM_SHARED((N_PAD, 16), jnp.float32),
        pltpu.SemaphoreType.DMA,
        pltpu.SemaphoreType.DMA,
        pltpu.SemaphoreType.DMA,
        pltpu.SemaphoreType.DMA,
    ],
    compiler_params=pltpu.CompilerParams(use_tc_tiling_on_sc=False),
)
def _sc_seg_counts(y_hbm, src_hbm, dst_hbm, z64_hbm, z16_hbm, ones_hbm,
                   out_hbm, cnt_hbm, src_v, dst_v, rows_v, ones_v,
                   acc_sh, cnt_sh, sem0, sem1, sem2, sem3):
    _seg_body(y_hbm, src_hbm, dst_hbm, z64_hbm, z16_hbm, ones_hbm,
              out_hbm, cnt_hbm, src_v, dst_v, rows_v, ones_v,
              acc_sh, cnt_sh, [sem0, sem1, sem2, sem3],
              NCHUNK1, with_counts=True)


@functools.partial(
    pl.kernel,
    out_type=jax.ShapeDtypeStruct((NC, N_PAD, D_HID), jnp.float32),
    mesh=_mesh,
    scratch_types=[
        pltpu.VMEM((NCHUNK2, GROUP2), jnp.int32),
        pltpu.VMEM((NCHUNK2, GROUP2), jnp.int32),
        pltpu.VMEM((2, GROUP2, D_HID), jnp.float32),
        pltpu.VMEM_SHARED((N_PAD, D_HID), jnp.float32),
        pltpu.SemaphoreType.DMA,
        pltpu.SemaphoreType.DMA,
        pltpu.SemaphoreType.DMA,
    ],
    compiler_params=pltpu.CompilerParams(use_tc_tiling_on_sc=False),
)
def _sc_seg(y_hbm, src_hbm, dst_hbm, z64_hbm, out_hbm,
            src_v, dst_v, rows_v, acc_sh, sem0, sem1, sem2):
    _seg_body(y_hbm, src_hbm, dst_hbm, z64_hbm, None, None,
              out_hbm, None, src_v, dst_v, rows_v, None,
              acc_sh, None, [sem0, sem1, sem2, None],
              NCHUNK2, with_counts=False)


def _tc_a_body(x_ref, wl1_ref, wr1_ref, bl1_ref, y1_ref, r1_ref):
    x = x_ref[...]
    dn = (((1,), (1,)), ((), ()))
    y1_ref[...] = lax.dot_general(x, wl1_ref[...], dn,
                                  preferred_element_type=jnp.float32)
    r1_ref[...] = lax.dot_general(x, wr1_ref[...], dn,
                                  preferred_element_type=jnp.float32) + bl1_ref[...]


_tc_a = pl.pallas_call(
    _tc_a_body,
    out_shape=(jax.ShapeDtypeStruct((N_NODES, D_HID), jnp.float32),
               jax.ShapeDtypeStruct((N_NODES, D_HID), jnp.float32)),
)


def _tc_b_body(s1_ref, cnt_ref, r1_ref, wr2_ref, bl2_ref, h_ref, r2_ref):
    s = (s1_ref[0] + s1_ref[1])[:N_NODES]
    c = (cnt_ref[0, :, 0:1] + cnt_ref[1, :, 0:1])[:N_NODES]
    agg = s / jnp.maximum(c, 1.0)
    h = jnp.maximum(agg + r1_ref[...], 0.0)
    h_ref[...] = h
    dn = (((1,), (1,)), ((), ()))
    r2_ref[...] = lax.dot_general(h, wr2_ref[...], dn,
                                  preferred_element_type=jnp.float32) + bl2_ref[...]


_tc_b = pl.pallas_call(
    _tc_b_body,
    out_shape=(jax.ShapeDtypeStruct((N_NODES, D_HID), jnp.float32),
               jax.ShapeDtypeStruct((N_NODES, D_OUT), jnp.float32)),
)


def _tc_c_body(s2_ref, cnt_ref, r2_ref, wl2_ref, out_ref):
    s = (s2_ref[0] + s2_ref[1])[:N_NODES]
    c = (cnt_ref[0, :, 0:1] + cnt_ref[1, :, 0:1])[:N_NODES]
    agg = s / jnp.maximum(c, 1.0)
    dn = (((1,), (1,)), ((), ()))
    z = lax.dot_general(agg, wl2_ref[...], dn,
                        preferred_element_type=jnp.float32) + r2_ref[...]
    m = jnp.max(z, axis=1, keepdims=True)
    lse = jnp.log(jnp.sum(jnp.exp(z - m), axis=1, keepdims=True)) + m
    out_ref[...] = z - lse


_tc_c = pl.pallas_call(
    _tc_c_body,
    out_shape=jax.ShapeDtypeStruct((N_NODES, D_OUT), jnp.float32),
)


def kernel(x, edge_index, Wl1, bl1, Wr1, Wl2, bl2, Wr2):
    ei = edge_index.astype(jnp.int32)
    src1 = ei[0].reshape(NW, NCHUNK1, GROUP1)
    dst1 = ei[1].reshape(NW, NCHUNK1, GROUP1)
    src2 = ei[0].reshape(NW, NCHUNK2, GROUP2)
    dst2 = ei[1].reshape(NW, NCHUNK2, GROUP2)
    z64 = jnp.zeros((ROWS_PER_SUB, D_HID), jnp.float32)
    z16 = jnp.zeros((ROWS_PER_SUB, 16), jnp.float32)
    ones16 = jnp.ones((GROUP1, 16), jnp.float32)

    y1, r1 = _tc_a(x, Wl1, Wr1, bl1.reshape(1, D_HID))
    s1, cntw = _sc_seg_counts(y1, src1, dst1, z64, z16, ones16)
    h, r2 = _tc_b(s1, cntw, r1, Wr2, bl2.reshape(1, D_OUT))
    s2 = _sc_seg(h, src2, dst2, z64)
    return _tc_c(s2, cntw, r2, Wl2)
